# baseline (speedup 1.0000x reference)
import numpy as np
import jax
import jax.numpy as jnp
from jax.experimental import pallas as pl
from jax.experimental.pallas import tpu as pltpu

_N = 256
_K = 33
_V = 64
_CB = 1024
_R = 1024
_ROWS = 4 * 4 * 1024

_BR_C = np.fft.rfft(np.eye(_N), norm='ortho')[:, :_K]
_BRR = np.ascontiguousarray(_BR_C.real, dtype=np.float32)
_BRI = np.ascontiguousarray(_BR_C.imag, dtype=np.float32)
_BRRT = np.ascontiguousarray(_BR_C.real.T, dtype=np.float32)
_BRIT = np.ascontiguousarray(_BR_C.imag.T, dtype=np.float32)
_CR = np.asarray(np.fft.irfft(np.eye(_K), n=_V, norm='ortho'),
                 dtype=np.float32)
_CI = np.asarray(np.fft.irfft(1j * np.eye(_K), n=_V, norm='ortho'),
                 dtype=np.float32)

_HI = jax.lax.Precision.HIGHEST


def _vq_kernel(x_ref, p_ref, cb_ref, brr_ref, bri_ref,
               brrt_ref, brit_ref, cr_ref, ci_ref, out_ref,
               m_ref, cbn_ref, fa_ref):
    i = pl.program_id(0)

    @pl.when(i == 0)
    def _init():
        p = p_ref[...]
        pr = jnp.dot(brrt_ref[...], p, precision=_HI)
        pi = jnp.dot(brit_ref[...], p, precision=_HI)
        cr = cr_ref[...]
        ci = ci_ref[...]
        d1 = pr * cr + pi * ci
        d2 = pr * ci - pi * cr
        m_ref[...] = (jnp.dot(brr_ref[...], d1, precision=_HI)
                      + jnp.dot(bri_ref[...], d2, precision=_HI))
        cb = cb_ref[...]
        cbn_ref[...] = cb / jnp.sqrt(jnp.sum(cb * cb, axis=1, keepdims=True))

    m = m_ref[...]
    cbn = cbn_ref[...]
    cdims = (((1,), (0,)), ((), ()))

    fv = fa_ref[...]
    sim = jax.lax.dot_general(cbn, fv, cdims)
    idx = jnp.argmax(sim, axis=0).astype(jnp.int32)

    ft = jax.lax.dot_general(m, x_ref[...],
                             (((0,), (1,)), ((), ())),
                             precision=_HI)
    fa_ref[...] = ft / jnp.sqrt(jnp.sum(ft * ft, axis=0, keepdims=True))

    out_ref[...] = idx.reshape(1, 1, 2 * _R)


def kernel(x, projector, codebook):
    xf = x.reshape(_ROWS, _N)
    p2 = projector.reshape(_N, 1)
    npair = _ROWS // (2 * _R)
    full = lambda shape: pl.BlockSpec(shape, lambda i: (0,) * len(shape))
    xspec = pl.BlockSpec((2 * _R, _N), lambda i: (jnp.minimum(i, npair - 1), 0))
    out = pl.pallas_call(
        _vq_kernel,
        grid=(npair + 1,),
        in_specs=[
            xspec,
            full((_N, 1)),
            full((_CB, _V)),
            full((_N, _K)),
            full((_N, _K)),
            full((_K, _N)),
            full((_K, _N)),
            full((_K, _V)),
            full((_K, _V)),
        ],
        out_specs=pl.BlockSpec((1, 1, 2 * _R),
                               lambda i: (jnp.maximum(i - 1, 0), 0, 0)),
        out_shape=jax.ShapeDtypeStruct((npair, 1, 2 * _R), jnp.int32),
        scratch_shapes=[
            pltpu.VMEM((_N, _V), jnp.float32),
            pltpu.VMEM((_CB, _V), jnp.float32),
            pltpu.VMEM((_V, 2 * _R), jnp.float32),
        ],
        compiler_params=pltpu.CompilerParams(
            dimension_semantics=("arbitrary",)),
    )(xf, p2, codebook, _BRR, _BRI, _BRRT, _BRIT, _CR, _CI)
    return out.reshape(x.shape[:-1])
